# Initial kernel scaffold; baseline (speedup 1.0000x reference)
import functools

import jax
import jax.numpy as jnp
from jax import lax
from jax.experimental import pallas as pl
from jax.experimental.pallas import tpu as pltpu
from jax.experimental.pallas import tpu_sc as plsc

_N = 10000
_D = 128
_E = 320000
_NP = 10240
_EP = 327680
_NC = 2
_NS = 16
_NW = _NC * _NS
_K = 128
_CH = _EP // (_NW * _K)
_RPS = _NP // _NS

_mesh = plsc.VectorSubcoreMesh(core_axis_name="c", subcore_axis_name="s")


@functools.partial(
    pl.kernel,
    out_type=(jax.ShapeDtypeStruct((_NC, _NP), jnp.float32),
              jax.ShapeDtypeStruct((_NC, _NP), jnp.float32)),
    mesh=_mesh,
    scratch_types=[
        pltpu.VMEM((_CH, _K), jnp.int32),
        pltpu.VMEM((_CH, _K), jnp.int32),
        pltpu.VMEM((_K,), jnp.float32),
        pltpu.VMEM((_RPS,), jnp.float32),
        pltpu.VMEM_SHARED((_NP,), jnp.float32),
        pltpu.VMEM_SHARED((_NP,), jnp.float32),
    ],
)
def _deg_kernel(src_hbm, dst_hbm, dout_hbm, din_hbm,
                src_v, dst_v, ones_v, zb_v, dout_sh, din_sh):
    c = lax.axis_index("c")
    s = lax.axis_index("s")
    wid = s * _NC + c
    base = s * _RPS

    @pl.loop(0, _RPS // 16)
    def _(i):
        zb_v[pl.ds(i * 16, 16)] = jnp.zeros((16,), jnp.float32)

    @pl.loop(0, _K // 16)
    def _(i):
        ones_v[pl.ds(i * 16, 16)] = jnp.ones((16,), jnp.float32)

    pltpu.sync_copy(zb_v, dout_sh.at[pl.ds(base, _RPS)])
    pltpu.sync_copy(zb_v, din_sh.at[pl.ds(base, _RPS)])
    plsc.subcore_barrier()

    pltpu.sync_copy(src_hbm.at[wid], src_v)
    pltpu.sync_copy(dst_hbm.at[wid], dst_v)

    @pl.loop(0, _CH)
    def _(j):
        pltpu.sync_copy(ones_v, dout_sh.at[src_v.at[j]], add=True)
        pltpu.sync_copy(ones_v, din_sh.at[dst_v.at[j]], add=True)

    plsc.subcore_barrier()
    pltpu.sync_copy(dout_sh.at[pl.ds(base, _RPS)],
                    dout_hbm.at[c, pl.ds(base, _RPS)])
    pltpu.sync_copy(din_sh.at[pl.ds(base, _RPS)],
                    din_hbm.at[c, pl.ds(base, _RPS)])


@functools.partial(
    pl.kernel,
    out_type=jax.ShapeDtypeStruct((_NC, _NP, _D), jnp.float32),
    mesh=_mesh,
    scratch_types=[
        pltpu.VMEM((_CH, _K), jnp.int32),
        pltpu.VMEM((_CH, _K), jnp.int32),
        pltpu.VMEM((_K, _D), jnp.float32),
        pltpu.VMEM((64, _D), jnp.float32),
        pltpu.VMEM_SHARED((_NP, _D), jnp.float32),
        pltpu.SemaphoreType.DMA,
    ],
)
def _scatter_kernel(h_hbm, src_hbm, dst_hbm, out_hbm,
                    src_v, dst_v, rows_v, zb_v, agg_sh, sem):
    c = lax.axis_index("c")
    s = lax.axis_index("s")
    wid = s * _NC + c
    base = s * _RPS

    @pl.loop(0, 64)
    def _(r):
        for q in range(_D // 16):
            zb_v[r, pl.ds(q * 16, 16)] = jnp.zeros((16,), jnp.float32)

    @pl.loop(0, _RPS // 64)
    def _(t):
        pltpu.sync_copy(zb_v, agg_sh.at[pl.ds(base + t * 64, 64)])
    plsc.subcore_barrier()

    pltpu.sync_copy(src_hbm.at[wid], src_v)
    pltpu.sync_copy(dst_hbm.at[wid], dst_v)

    @pl.loop(0, _CH)
    def _(j):
        pltpu.async_copy(h_hbm.at[src_v.at[j]], rows_v, sem).wait()
        pltpu.sync_copy(rows_v, agg_sh.at[dst_v.at[j]], add=True)

    plsc.subcore_barrier()
    pltpu.sync_copy(agg_sh.at[pl.ds(base, _RPS)],
                    out_hbm.at[c, pl.ds(base, _RPS)])


_R = 1024
_G = _NP // _R


def _norm(deg):
    return jnp.where(deg > 0, lax.rsqrt(jnp.maximum(deg, 1e-12)), 0.0)


def _prep_body(x_ref, do0_ref, do1_ref, di0_ref, di1_ref,
               h0_ref, ns_ref, nd_ref):
    ns = _norm(do0_ref[...] + do1_ref[...])
    nd = _norm(di0_ref[...] + di1_ref[...])
    h0_ref[...] = x_ref[...] * ns
    ns_ref[...] = ns
    nd_ref[...] = nd


_prep = pl.pallas_call(
    _prep_body,
    grid=(_G,),
    in_specs=[pl.BlockSpec((_R, _D), lambda i: (i, 0))] +
             [pl.BlockSpec((_R, 1), lambda i: (i, 0))] * 4,
    out_specs=[pl.BlockSpec((_R, _D), lambda i: (i, 0)),
               pl.BlockSpec((_R, 1), lambda i: (i, 0)),
               pl.BlockSpec((_R, 1), lambda i: (i, 0))],
    out_shape=[jax.ShapeDtypeStruct((_NP, _D), jnp.float32),
               jax.ShapeDtypeStruct((_NP, 1), jnp.float32),
               jax.ShapeDtypeStruct((_NP, 1), jnp.float32)],
)


def _layer_body(p0_ref, p1_ref, nd_ref, ns_ref, w_ref, b_ref, out_ref):
    agg = (p0_ref[...] + p1_ref[...]) * nd_ref[...]
    h = jnp.dot(agg, w_ref[...], preferred_element_type=jnp.float32)
    out_ref[...] = jnp.maximum(h + b_ref[...], 0.0) * ns_ref[...]


_layer1 = pl.pallas_call(
    _layer_body,
    grid=(_G,),
    in_specs=[pl.BlockSpec((_R, _D), lambda i: (i, 0)),
              pl.BlockSpec((_R, _D), lambda i: (i, 0)),
              pl.BlockSpec((_R, 1), lambda i: (i, 0)),
              pl.BlockSpec((_R, 1), lambda i: (i, 0)),
              pl.BlockSpec((_D, _D), lambda i: (0, 0)),
              pl.BlockSpec((1, _D), lambda i: (0, 0))],
    out_specs=pl.BlockSpec((_R, _D), lambda i: (i, 0)),
    out_shape=jax.ShapeDtypeStruct((_NP, _D), jnp.float32),
)


def _final_body(p0_ref, p1_ref, nd_ref, w_ref, b_ref, out_ref):
    i = pl.program_id(0)
    agg = (p0_ref[...] + p1_ref[...]) * nd_ref[...]
    h = jnp.dot(agg, w_ref[...], preferred_element_type=jnp.float32)
    h = jnp.maximum(h + b_ref[...], 0.0)
    row = lax.broadcasted_iota(jnp.int32, (_R, 1), 0) + i * _R
    h = jnp.where(row < _N, h, 0.0)
    part = jnp.sum(h, axis=0, keepdims=True) * (1.0 / _N)

    @pl.when(i == 0)
    def _():
        out_ref[...] = part

    @pl.when(i > 0)
    def _():
        out_ref[...] += part


_final = pl.pallas_call(
    _final_body,
    grid=(_G,),
    in_specs=[pl.BlockSpec((_R, _D), lambda i: (i, 0)),
              pl.BlockSpec((_R, _D), lambda i: (i, 0)),
              pl.BlockSpec((_R, 1), lambda i: (i, 0)),
              pl.BlockSpec((_D, _D), lambda i: (0, 0)),
              pl.BlockSpec((1, _D), lambda i: (0, 0))],
    out_specs=pl.BlockSpec((1, _D), lambda i: (0, 0)),
    out_shape=jax.ShapeDtypeStruct((1, _D), jnp.float32),
)


def kernel(features, edge_index, W1, b1, W2, b2):
    src = edge_index[0].astype(jnp.int32)
    dst = edge_index[1].astype(jnp.int32)
    pad = jnp.full((_EP - _E,), _NP - 1, jnp.int32)
    srcp = jnp.concatenate([src, pad]).reshape(_NW, _CH, _K)
    dstp = jnp.concatenate([dst, pad]).reshape(_NW, _CH, _K)
    xp = jnp.pad(features, ((0, _NP - _N), (0, 0)))

    dout_p, din_p = _deg_kernel(srcp, dstp)
    h0, ns, nd = _prep(xp,
                       dout_p[0].reshape(_NP, 1), dout_p[1].reshape(_NP, 1),
                       din_p[0].reshape(_NP, 1), din_p[1].reshape(_NP, 1))

    agg1 = _scatter_kernel(h0, srcp, dstp)
    h1 = _layer1(agg1[0], agg1[1], nd, ns, W1, b1.reshape(1, _D))

    agg2 = _scatter_kernel(h1, srcp, dstp)
    return _final(agg2[0], agg2[1], nd, W2, b2.reshape(1, _D))


# speedup vs baseline: 3.4379x; 3.4379x over previous
import functools

import jax
import jax.numpy as jnp
from jax import lax
from jax.experimental import pallas as pl
from jax.experimental.pallas import tpu as pltpu
from jax.experimental.pallas import tpu_sc as plsc

_N = 10000
_D = 128
_E = 320000
_NP = 10240
_EP = 327680
_NC = 2
_NS = 16
_NW = _NC * _NS
_K = 128
_CH = _EP // (_NW * _K)
_RPS = _NP // _NS

def _deg_body(src_hbm, dst_hbm, dout_hbm, din_hbm,
              src_v, dst_v, ones_v, zb_v, dout_sh, din_sh):
    c = lax.axis_index("c")
    s = lax.axis_index("s")
    wid = s * _NC + c
    base = s * _RPS

    @pl.loop(0, _RPS // 16)
    def _(i):
        zb_v[pl.ds(i * 16, 16)] = jnp.zeros((16,), jnp.float32)

    @pl.loop(0, _K // 16)
    def _(i):
        ones_v[pl.ds(i * 16, 16)] = jnp.ones((16,), jnp.float32)

    pltpu.sync_copy(zb_v, dout_sh.at[pl.ds(base, _RPS)])
    pltpu.sync_copy(zb_v, din_sh.at[pl.ds(base, _RPS)])
    plsc.subcore_barrier()

    pltpu.sync_copy(src_hbm.at[wid], src_v)
    pltpu.sync_copy(dst_hbm.at[wid], dst_v)

    @pl.loop(0, _CH)
    def _(j):
        pltpu.sync_copy(ones_v, dout_sh.at[src_v.at[j]], add=True)
        pltpu.sync_copy(ones_v, din_sh.at[dst_v.at[j]], add=True)

    plsc.subcore_barrier()
    pltpu.sync_copy(dout_sh.at[pl.ds(base, _RPS)],
                    dout_hbm.at[c, pl.ds(base, _RPS)])
    pltpu.sync_copy(din_sh.at[pl.ds(base, _RPS)],
                    din_hbm.at[c, pl.ds(base, _RPS)])


def _scatter_body(h_hbm, src_hbm, dst_hbm, out_hbm,
                  src_v, dst_v, rows_v, zb_v, agg_sh, sem):
    c = lax.axis_index("c")
    s = lax.axis_index("s")
    wid = s * _NC + c
    base = s * _RPS

    @pl.loop(0, 64)
    def _(r):
        for q in range(_D // 16):
            zb_v[r, pl.ds(q * 16, 16)] = jnp.zeros((16,), jnp.float32)

    @pl.loop(0, _RPS // 64)
    def _(t):
        pltpu.sync_copy(zb_v, agg_sh.at[pl.ds(base + t * 64, 64)])
    plsc.subcore_barrier()

    pltpu.sync_copy(src_hbm.at[wid], src_v)
    pltpu.sync_copy(dst_hbm.at[wid], dst_v)

    @pl.loop(0, _CH)
    def _(j):
        pltpu.async_copy(h_hbm.at[src_v.at[j]], rows_v, sem).wait()
        pltpu.sync_copy(rows_v, agg_sh.at[dst_v.at[j]], add=True)

    plsc.subcore_barrier()
    pltpu.sync_copy(agg_sh.at[pl.ds(base, _RPS)],
                    out_hbm.at[c, pl.ds(base, _RPS)])


@functools.cache
def _sc_kernels():
    mesh = plsc.VectorSubcoreMesh(core_axis_name="c", subcore_axis_name="s",
                                  num_cores=_NC, num_subcores=_NS)
    deg = pl.kernel(
        _deg_body,
        out_type=(jax.ShapeDtypeStruct((_NC, _NP), jnp.float32),
                  jax.ShapeDtypeStruct((_NC, _NP), jnp.float32)),
        mesh=mesh,
        scratch_types=[
            pltpu.VMEM((_CH, _K), jnp.int32),
            pltpu.VMEM((_CH, _K), jnp.int32),
            pltpu.VMEM((_K,), jnp.float32),
            pltpu.VMEM((_RPS,), jnp.float32),
            pltpu.VMEM_SHARED((_NP,), jnp.float32),
            pltpu.VMEM_SHARED((_NP,), jnp.float32),
        ],
    )
    scatter = pl.kernel(
        _scatter_body,
        out_type=jax.ShapeDtypeStruct((_NC, _NP, _D), jnp.float32),
        mesh=mesh,
        scratch_types=[
            pltpu.VMEM((_CH, _K), jnp.int32),
            pltpu.VMEM((_CH, _K), jnp.int32),
            pltpu.VMEM((_K, _D), jnp.float32),
            pltpu.VMEM((64, _D), jnp.float32),
            pltpu.VMEM_SHARED((_NP, _D), jnp.float32),
            pltpu.SemaphoreType.DMA,
        ],
    )
    return deg, scatter


_R = 1024
_G = _NP // _R


def _norm(deg):
    return jnp.where(deg > 0, lax.rsqrt(jnp.maximum(deg, 1e-12)), 0.0)


def _prep_body(x_ref, do0_ref, do1_ref, di0_ref, di1_ref,
               h0_ref, ns_ref, nd_ref):
    ns = _norm(do0_ref[...] + do1_ref[...])
    nd = _norm(di0_ref[...] + di1_ref[...])
    h0_ref[...] = x_ref[...] * ns
    ns_ref[...] = ns
    nd_ref[...] = nd


_prep = pl.pallas_call(
    _prep_body,
    grid=(_G,),
    in_specs=[pl.BlockSpec((_R, _D), lambda i: (i, 0))] +
             [pl.BlockSpec((_R, 1), lambda i: (i, 0))] * 4,
    out_specs=[pl.BlockSpec((_R, _D), lambda i: (i, 0)),
               pl.BlockSpec((_R, 1), lambda i: (i, 0)),
               pl.BlockSpec((_R, 1), lambda i: (i, 0))],
    out_shape=[jax.ShapeDtypeStruct((_NP, _D), jnp.float32),
               jax.ShapeDtypeStruct((_NP, 1), jnp.float32),
               jax.ShapeDtypeStruct((_NP, 1), jnp.float32)],
)


def _layer_body(p0_ref, p1_ref, nd_ref, ns_ref, w_ref, b_ref, out_ref):
    agg = (p0_ref[...] + p1_ref[...]) * nd_ref[...]
    h = jnp.dot(agg, w_ref[...], preferred_element_type=jnp.float32)
    out_ref[...] = jnp.maximum(h + b_ref[...], 0.0) * ns_ref[...]


_layer1 = pl.pallas_call(
    _layer_body,
    grid=(_G,),
    in_specs=[pl.BlockSpec((_R, _D), lambda i: (i, 0)),
              pl.BlockSpec((_R, _D), lambda i: (i, 0)),
              pl.BlockSpec((_R, 1), lambda i: (i, 0)),
              pl.BlockSpec((_R, 1), lambda i: (i, 0)),
              pl.BlockSpec((_D, _D), lambda i: (0, 0)),
              pl.BlockSpec((1, _D), lambda i: (0, 0))],
    out_specs=pl.BlockSpec((_R, _D), lambda i: (i, 0)),
    out_shape=jax.ShapeDtypeStruct((_NP, _D), jnp.float32),
)


def _final_body(p0_ref, p1_ref, nd_ref, w_ref, b_ref, out_ref):
    i = pl.program_id(0)
    agg = (p0_ref[...] + p1_ref[...]) * nd_ref[...]
    h = jnp.dot(agg, w_ref[...], preferred_element_type=jnp.float32)
    h = jnp.maximum(h + b_ref[...], 0.0)
    row = lax.broadcasted_iota(jnp.int32, (_R, 1), 0) + i * _R
    h = jnp.where(row < _N, h, 0.0)
    part = jnp.sum(h, axis=0, keepdims=True) * (1.0 / _N)

    @pl.when(i == 0)
    def _():
        out_ref[...] = part

    @pl.when(i > 0)
    def _():
        out_ref[...] += part


_final = pl.pallas_call(
    _final_body,
    grid=(_G,),
    in_specs=[pl.BlockSpec((_R, _D), lambda i: (i, 0)),
              pl.BlockSpec((_R, _D), lambda i: (i, 0)),
              pl.BlockSpec((_R, 1), lambda i: (i, 0)),
              pl.BlockSpec((_D, _D), lambda i: (0, 0)),
              pl.BlockSpec((1, _D), lambda i: (0, 0))],
    out_specs=pl.BlockSpec((1, _D), lambda i: (0, 0)),
    out_shape=jax.ShapeDtypeStruct((1, _D), jnp.float32),
)


def kernel(features, edge_index, W1, b1, W2, b2):
    src = edge_index[0].astype(jnp.int32)
    dst = edge_index[1].astype(jnp.int32)
    pad = jnp.full((_EP - _E,), _NP - 1, jnp.int32)
    srcp = jnp.concatenate([src, pad]).reshape(_NW, _CH, _K)
    dstp = jnp.concatenate([dst, pad]).reshape(_NW, _CH, _K)
    xp = jnp.pad(features, ((0, _NP - _N), (0, 0)))

    _deg_kernel, _scatter_kernel = _sc_kernels()
    dout_p, din_p = _deg_kernel(srcp, dstp)
    h0, ns, nd = _prep(xp,
                       dout_p[0].reshape(_NP, 1), dout_p[1].reshape(_NP, 1),
                       din_p[0].reshape(_NP, 1), din_p[1].reshape(_NP, 1))

    agg1 = _scatter_kernel(h0, srcp, dstp)
    h1 = _layer1(agg1[0], agg1[1], nd, ns, W1, b1.reshape(1, _D))

    agg2 = _scatter_kernel(h1, srcp, dstp)
    return _final(agg2[0], agg2[1], nd, W2, b2.reshape(1, _D))


# speedup vs baseline: 3.7649x; 1.0951x over previous
import functools

import jax
import jax.numpy as jnp
from jax import lax
from jax.experimental import pallas as pl
from jax.experimental.pallas import tpu as pltpu
from jax.experimental.pallas import tpu_sc as plsc

_N = 10000
_D = 128
_E = 320000
_NP = 10240
_EP = 327680
_NC = 2
_NS = 16
_NW = _NC * _NS
_K = 128
_CH = _EP // (_NW * _K)
_RPS = _NP // _NS

def _deg_body(src_hbm, dst_hbm, dout_hbm, din_hbm,
              src_v, dst_v, ones_v, zb_v, dout_sh, din_sh):
    c = lax.axis_index("c")
    s = lax.axis_index("s")
    wid = s * _NC + c
    base = s * _RPS

    @pl.loop(0, _RPS // 16)
    def _(i):
        zb_v[pl.ds(i * 16, 16)] = jnp.zeros((16,), jnp.float32)

    @pl.loop(0, _K // 16)
    def _(i):
        ones_v[pl.ds(i * 16, 16)] = jnp.ones((16,), jnp.float32)

    pltpu.sync_copy(zb_v, dout_sh.at[pl.ds(base, _RPS)])
    pltpu.sync_copy(zb_v, din_sh.at[pl.ds(base, _RPS)])
    plsc.subcore_barrier()

    pltpu.sync_copy(src_hbm.at[wid], src_v)
    pltpu.sync_copy(dst_hbm.at[wid], dst_v)

    @pl.loop(0, _CH)
    def _(j):
        pltpu.sync_copy(ones_v, dout_sh.at[src_v.at[j]], add=True)
        pltpu.sync_copy(ones_v, din_sh.at[dst_v.at[j]], add=True)

    plsc.subcore_barrier()
    pltpu.sync_copy(dout_sh.at[pl.ds(base, _RPS)],
                    dout_hbm.at[c, pl.ds(base, _RPS)])
    pltpu.sync_copy(din_sh.at[pl.ds(base, _RPS)],
                    din_hbm.at[c, pl.ds(base, _RPS)])


def _scatter_body(h_hbm, src_hbm, dst_hbm, out_hbm,
                  src_v, dst_v, rows_a, rows_b, agg_sh, sem_a, sem_b):
    c = lax.axis_index("c")
    s = lax.axis_index("s")
    wid = s * _NC + c
    base = s * _RPS
    hch = _CH // 2

    @pl.loop(0, _K)
    def _(r):
        for q in range(_D // 16):
            rows_a[r, pl.ds(q * 16, 16)] = jnp.zeros((16,), jnp.float32)

    @pl.loop(0, _RPS // _K)
    def _(t):
        pltpu.sync_copy(rows_a, agg_sh.at[pl.ds(base + t * _K, _K)])
    plsc.subcore_barrier()

    def _issue(j, buf, sem):
        pltpu.async_copy(h_hbm.at[src_v.at[j]], buf, sem)

    def _drain(buf, sem):
        pltpu.make_async_copy(h_hbm.at[pl.ds(0, _K)], buf, sem).wait()

    @pl.loop(0, 2)
    def _(p):
        pltpu.sync_copy(src_hbm.at[wid, pl.ds(p * hch, hch)], src_v)
        pltpu.sync_copy(dst_hbm.at[wid, pl.ds(p * hch, hch)], dst_v)
        _issue(0, rows_a, sem_a)

        @pl.loop(0, hch, step=2)
        def _(j):
            _drain(rows_a, sem_a)
            _issue(j + 1, rows_b, sem_b)
            pltpu.sync_copy(rows_a, agg_sh.at[dst_v.at[j]], add=True)

            @pl.when(j + 2 < hch)
            def _():
                _issue(j + 2, rows_a, sem_a)

            _drain(rows_b, sem_b)
            pltpu.sync_copy(rows_b, agg_sh.at[dst_v.at[j + 1]], add=True)

    plsc.subcore_barrier()
    pltpu.sync_copy(agg_sh.at[pl.ds(base, _RPS)],
                    out_hbm.at[c, pl.ds(base, _RPS)])


@functools.cache
def _sc_kernels():
    mesh = plsc.VectorSubcoreMesh(core_axis_name="c", subcore_axis_name="s",
                                  num_cores=_NC, num_subcores=_NS)
    deg = pl.kernel(
        _deg_body,
        out_type=(jax.ShapeDtypeStruct((_NC, _NP), jnp.float32),
                  jax.ShapeDtypeStruct((_NC, _NP), jnp.float32)),
        mesh=mesh,
        scratch_types=[
            pltpu.VMEM((_CH, _K), jnp.int32),
            pltpu.VMEM((_CH, _K), jnp.int32),
            pltpu.VMEM((_K,), jnp.float32),
            pltpu.VMEM((_RPS,), jnp.float32),
            pltpu.VMEM_SHARED((_NP,), jnp.float32),
            pltpu.VMEM_SHARED((_NP,), jnp.float32),
        ],
    )
    scatter = pl.kernel(
        _scatter_body,
        out_type=jax.ShapeDtypeStruct((_NC, _NP, _D), jnp.float32),
        mesh=mesh,
        scratch_types=[
            pltpu.VMEM((_CH // 2, _K), jnp.int32),
            pltpu.VMEM((_CH // 2, _K), jnp.int32),
            pltpu.VMEM((_K, _D), jnp.float32),
            pltpu.VMEM((_K, _D), jnp.float32),
            pltpu.VMEM_SHARED((_NP, _D), jnp.float32),
            pltpu.SemaphoreType.DMA,
            pltpu.SemaphoreType.DMA,
        ],
    )
    return deg, scatter


_R = 1024
_G = _NP // _R


def _norm(deg):
    return jnp.where(deg > 0, lax.rsqrt(jnp.maximum(deg, 1e-12)), 0.0)


def _prep_body(x_ref, do0_ref, do1_ref, di0_ref, di1_ref,
               h0_ref, ns_ref, nd_ref):
    ns = _norm(do0_ref[...] + do1_ref[...])
    nd = _norm(di0_ref[...] + di1_ref[...])
    h0_ref[...] = x_ref[...] * ns
    ns_ref[...] = ns
    nd_ref[...] = nd


_prep = pl.pallas_call(
    _prep_body,
    grid=(_G,),
    in_specs=[pl.BlockSpec((_R, _D), lambda i: (i, 0))] +
             [pl.BlockSpec((_R, 1), lambda i: (i, 0))] * 4,
    out_specs=[pl.BlockSpec((_R, _D), lambda i: (i, 0)),
               pl.BlockSpec((_R, 1), lambda i: (i, 0)),
               pl.BlockSpec((_R, 1), lambda i: (i, 0))],
    out_shape=[jax.ShapeDtypeStruct((_NP, _D), jnp.float32),
               jax.ShapeDtypeStruct((_NP, 1), jnp.float32),
               jax.ShapeDtypeStruct((_NP, 1), jnp.float32)],
)


def _layer_body(p0_ref, p1_ref, nd_ref, ns_ref, w_ref, b_ref, out_ref):
    agg = (p0_ref[...] + p1_ref[...]) * nd_ref[...]
    h = jnp.dot(agg, w_ref[...], preferred_element_type=jnp.float32)
    out_ref[...] = jnp.maximum(h + b_ref[...], 0.0) * ns_ref[...]


_layer1 = pl.pallas_call(
    _layer_body,
    grid=(_G,),
    in_specs=[pl.BlockSpec((_R, _D), lambda i: (i, 0)),
              pl.BlockSpec((_R, _D), lambda i: (i, 0)),
              pl.BlockSpec((_R, 1), lambda i: (i, 0)),
              pl.BlockSpec((_R, 1), lambda i: (i, 0)),
              pl.BlockSpec((_D, _D), lambda i: (0, 0)),
              pl.BlockSpec((1, _D), lambda i: (0, 0))],
    out_specs=pl.BlockSpec((_R, _D), lambda i: (i, 0)),
    out_shape=jax.ShapeDtypeStruct((_NP, _D), jnp.float32),
)


def _final_body(p0_ref, p1_ref, nd_ref, w_ref, b_ref, out_ref):
    i = pl.program_id(0)
    agg = (p0_ref[...] + p1_ref[...]) * nd_ref[...]
    h = jnp.dot(agg, w_ref[...], preferred_element_type=jnp.float32)
    h = jnp.maximum(h + b_ref[...], 0.0)
    row = lax.broadcasted_iota(jnp.int32, (_R, 1), 0) + i * _R
    h = jnp.where(row < _N, h, 0.0)
    part = jnp.sum(h, axis=0, keepdims=True) * (1.0 / _N)

    @pl.when(i == 0)
    def _():
        out_ref[...] = part

    @pl.when(i > 0)
    def _():
        out_ref[...] += part


_final = pl.pallas_call(
    _final_body,
    grid=(_G,),
    in_specs=[pl.BlockSpec((_R, _D), lambda i: (i, 0)),
              pl.BlockSpec((_R, _D), lambda i: (i, 0)),
              pl.BlockSpec((_R, 1), lambda i: (i, 0)),
              pl.BlockSpec((_D, _D), lambda i: (0, 0)),
              pl.BlockSpec((1, _D), lambda i: (0, 0))],
    out_specs=pl.BlockSpec((1, _D), lambda i: (0, 0)),
    out_shape=jax.ShapeDtypeStruct((1, _D), jnp.float32),
)


def kernel(features, edge_index, W1, b1, W2, b2):
    src = edge_index[0].astype(jnp.int32)
    dst = edge_index[1].astype(jnp.int32)
    pad = jnp.full((_EP - _E,), _NP - 1, jnp.int32)
    srcp = jnp.concatenate([src, pad]).reshape(_NW, _CH, _K)
    dstp = jnp.concatenate([dst, pad]).reshape(_NW, _CH, _K)
    xp = jnp.pad(features, ((0, _NP - _N), (0, 0)))

    _deg_kernel, _scatter_kernel = _sc_kernels()
    dout_p, din_p = _deg_kernel(srcp, dstp)
    h0, ns, nd = _prep(xp,
                       dout_p[0].reshape(_NP, 1), dout_p[1].reshape(_NP, 1),
                       din_p[0].reshape(_NP, 1), din_p[1].reshape(_NP, 1))

    agg1 = _scatter_kernel(h0, srcp, dstp)
    h1 = _layer1(agg1[0], agg1[1], nd, ns, W1, b1.reshape(1, _D))

    agg2 = _scatter_kernel(h1, srcp, dstp)
    return _final(agg2[0], agg2[1], nd, W2, b2.reshape(1, _D))


# speedup vs baseline: 11.6215x; 3.0868x over previous
import functools

import jax
import jax.numpy as jnp
from jax import lax
from jax.experimental import pallas as pl
from jax.experimental.pallas import tpu as pltpu
from jax.experimental.pallas import tpu_sc as plsc

_N = 10000
_D = 128
_E = 320000
_NP = 10240
_EP = 327680
_NC = 2
_NS = 16
_NW = _NC * _NS
_K = 128
_CH = _EP // (_NW * _K)
_RPS = _NP // _NS

def _deg_body(src_hbm, dst_hbm, dout_hbm, din_hbm,
              src_v, dst_v, ones_v, zb_v, dout_sh, din_sh):
    c = lax.axis_index("c")
    s = lax.axis_index("s")
    wid = s * _NC + c
    base = s * _RPS

    @pl.loop(0, _RPS // 16)
    def _(i):
        zb_v[pl.ds(i * 16, 16)] = jnp.zeros((16,), jnp.float32)

    @pl.loop(0, _K // 16)
    def _(i):
        ones_v[pl.ds(i * 16, 16)] = jnp.ones((16,), jnp.float32)

    pltpu.sync_copy(zb_v, dout_sh.at[pl.ds(base, _RPS)])
    pltpu.sync_copy(zb_v, din_sh.at[pl.ds(base, _RPS)])
    plsc.subcore_barrier()

    pltpu.sync_copy(src_hbm.at[wid], src_v)
    pltpu.sync_copy(dst_hbm.at[wid], dst_v)

    @pl.loop(0, _CH)
    def _(j):
        pltpu.sync_copy(ones_v, dout_sh.at[src_v.at[j]], add=True)
        pltpu.sync_copy(ones_v, din_sh.at[dst_v.at[j]], add=True)

    plsc.subcore_barrier()
    pltpu.sync_copy(dout_sh.at[pl.ds(base, _RPS)],
                    dout_hbm.at[c, pl.ds(base, _RPS)])
    pltpu.sync_copy(din_sh.at[pl.ds(base, _RPS)],
                    din_hbm.at[c, pl.ds(base, _RPS)])


def _scatter_body(h_hbm, src_hbm, dst_hbm, out_hbm,
                  src_v, dst_v, rows_a, rows_b, agg_sh, sem_a, sem_b):
    c = lax.axis_index("c")
    s = lax.axis_index("s")
    wid = s * _NC + c
    base = s * _RPS
    hch = _CH // 2

    @pl.loop(0, _K)
    def _(r):
        for q in range(_D // 16):
            rows_a[r, pl.ds(q * 16, 16)] = jnp.zeros((16,), jnp.float32)

    @pl.loop(0, _RPS // _K)
    def _(t):
        pltpu.sync_copy(rows_a, agg_sh.at[pl.ds(base + t * _K, _K)])
    plsc.subcore_barrier()

    def _issue(j, buf, sem):
        pltpu.async_copy(h_hbm.at[src_v.at[j]], buf, sem)

    def _drain(buf, sem):
        pltpu.make_async_copy(h_hbm.at[pl.ds(0, _K)], buf, sem).wait()

    @pl.loop(0, 2)
    def _(p):
        pltpu.sync_copy(src_hbm.at[wid, pl.ds(p * hch, hch)], src_v)
        pltpu.sync_copy(dst_hbm.at[wid, pl.ds(p * hch, hch)], dst_v)
        _issue(0, rows_a, sem_a)

        @pl.loop(0, hch, step=2)
        def _(j):
            _drain(rows_a, sem_a)
            _issue(j + 1, rows_b, sem_b)
            pltpu.sync_copy(rows_a, agg_sh.at[dst_v.at[j]], add=True)

            @pl.when(j + 2 < hch)
            def _():
                _issue(j + 2, rows_a, sem_a)

            _drain(rows_b, sem_b)
            pltpu.sync_copy(rows_b, agg_sh.at[dst_v.at[j + 1]], add=True)

    plsc.subcore_barrier()
    pltpu.sync_copy(agg_sh.at[pl.ds(base, _RPS)],
                    out_hbm.at[c, pl.ds(base, _RPS)])


@functools.cache
def _sc_kernels():
    mesh = plsc.VectorSubcoreMesh(core_axis_name="c", subcore_axis_name="s",
                                  num_cores=_NC, num_subcores=_NS)
    deg = pl.kernel(
        _deg_body,
        out_type=(jax.ShapeDtypeStruct((_NC, _NP), jnp.float32),
                  jax.ShapeDtypeStruct((_NC, _NP), jnp.float32)),
        mesh=mesh,
        scratch_types=[
            pltpu.VMEM((_CH, _K), jnp.int32),
            pltpu.VMEM((_CH, _K), jnp.int32),
            pltpu.VMEM((_K,), jnp.float32),
            pltpu.VMEM((_RPS,), jnp.float32),
            pltpu.VMEM_SHARED((_NP,), jnp.float32),
            pltpu.VMEM_SHARED((_NP,), jnp.float32),
        ],
    )
    scatter = pl.kernel(
        _scatter_body,
        out_type=jax.ShapeDtypeStruct((_NC, _NP, _D), jnp.float32),
        mesh=mesh,
        scratch_types=[
            pltpu.VMEM((_CH // 2, _K), jnp.int32),
            pltpu.VMEM((_CH // 2, _K), jnp.int32),
            pltpu.VMEM((_K, _D), jnp.float32),
            pltpu.VMEM((_K, _D), jnp.float32),
            pltpu.VMEM_SHARED((_NP, _D), jnp.float32),
            pltpu.SemaphoreType.DMA,
            pltpu.SemaphoreType.DMA,
        ],
    )
    return deg, scatter


_R = 1024
_G = _NP // _R


def _norm(deg):
    return jnp.where(deg > 0, lax.rsqrt(jnp.maximum(deg, 1e-12)), 0.0)


def _prep_body(x_ref, do0_ref, do1_ref, di0_ref, di1_ref,
               h0_ref, ns_ref, nd_ref):
    ns = _norm(do0_ref[...] + do1_ref[...])
    nd = _norm(di0_ref[...] + di1_ref[...])
    h0_ref[...] = x_ref[...] * ns
    ns_ref[...] = ns
    nd_ref[...] = nd


_prep = pl.pallas_call(
    _prep_body,
    grid=(_G,),
    in_specs=[pl.BlockSpec((_R, _D), lambda i: (i, 0))] +
             [pl.BlockSpec((_R, 1), lambda i: (i, 0))] * 4,
    out_specs=[pl.BlockSpec((_R, _D), lambda i: (i, 0)),
               pl.BlockSpec((_R, 1), lambda i: (i, 0)),
               pl.BlockSpec((_R, 1), lambda i: (i, 0))],
    out_shape=[jax.ShapeDtypeStruct((_NP, _D), jnp.float32),
               jax.ShapeDtypeStruct((_NP, 1), jnp.float32),
               jax.ShapeDtypeStruct((_NP, 1), jnp.float32)],
)


def _layer_body(p0_ref, p1_ref, nd_ref, ns_ref, w_ref, b_ref, out_ref):
    agg = (p0_ref[...] + p1_ref[...]) * nd_ref[...]
    h = jnp.dot(agg, w_ref[...], preferred_element_type=jnp.float32)
    out_ref[...] = jnp.maximum(h + b_ref[...], 0.0) * ns_ref[...]


_layer1 = pl.pallas_call(
    _layer_body,
    grid=(_G,),
    in_specs=[pl.BlockSpec((_R, _D), lambda i: (i, 0)),
              pl.BlockSpec((_R, _D), lambda i: (i, 0)),
              pl.BlockSpec((_R, 1), lambda i: (i, 0)),
              pl.BlockSpec((_R, 1), lambda i: (i, 0)),
              pl.BlockSpec((_D, _D), lambda i: (0, 0)),
              pl.BlockSpec((1, _D), lambda i: (0, 0))],
    out_specs=pl.BlockSpec((_R, _D), lambda i: (i, 0)),
    out_shape=jax.ShapeDtypeStruct((_NP, _D), jnp.float32),
)


def _final_body(p0_ref, p1_ref, nd_ref, w_ref, b_ref, out_ref):
    i = pl.program_id(0)
    agg = (p0_ref[...] + p1_ref[...]) * nd_ref[...]
    h = jnp.dot(agg, w_ref[...], preferred_element_type=jnp.float32)
    h = jnp.maximum(h + b_ref[...], 0.0)
    row = lax.broadcasted_iota(jnp.int32, (_R, 1), 0) + i * _R
    h = jnp.where(row < _N, h, 0.0)
    part = jnp.sum(h, axis=0, keepdims=True) * (1.0 / _N)

    @pl.when(i == 0)
    def _():
        out_ref[...] = part

    @pl.when(i > 0)
    def _():
        out_ref[...] += part


_final = pl.pallas_call(
    _final_body,
    grid=(_G,),
    in_specs=[pl.BlockSpec((_R, _D), lambda i: (i, 0)),
              pl.BlockSpec((_R, _D), lambda i: (i, 0)),
              pl.BlockSpec((_R, 1), lambda i: (i, 0)),
              pl.BlockSpec((_D, _D), lambda i: (0, 0)),
              pl.BlockSpec((1, _D), lambda i: (0, 0))],
    out_specs=pl.BlockSpec((1, _D), lambda i: (0, 0)),
    out_shape=jax.ShapeDtypeStruct((1, _D), jnp.float32),
)


def kernel(features, edge_index, W1, b1, W2, b2):
    src = edge_index[0].astype(jnp.int32)
    dst = edge_index[1].astype(jnp.int32)
    pad = _N + (jnp.arange(_EP - _E, dtype=jnp.int32) % (_NP - _N))
    srcp = jnp.concatenate([src, pad]).reshape(_NW, _CH, _K)
    dstp = jnp.concatenate([dst, pad]).reshape(_NW, _CH, _K)
    xp = jnp.pad(features, ((0, _NP - _N), (0, 0)))

    _deg_kernel, _scatter_kernel = _sc_kernels()
    dout_p, din_p = _deg_kernel(srcp, dstp)
    h0, ns, nd = _prep(xp,
                       dout_p[0].reshape(_NP, 1), dout_p[1].reshape(_NP, 1),
                       din_p[0].reshape(_NP, 1), din_p[1].reshape(_NP, 1))

    agg1 = _scatter_kernel(h0, srcp, dstp)
    h1 = _layer1(agg1[0], agg1[1], nd, ns, W1, b1.reshape(1, _D))

    agg2 = _scatter_kernel(h1, srcp, dstp)
    return _final(agg2[0], agg2[1], nd, W2, b2.reshape(1, _D))


# speedup vs baseline: 12.0602x; 1.0377x over previous
import functools

import jax
import jax.numpy as jnp
from jax import lax
from jax.experimental import pallas as pl
from jax.experimental.pallas import tpu as pltpu
from jax.experimental.pallas import tpu_sc as plsc

_N = 10000
_D = 128
_E = 320000
_NP = 10240
_EP = 327680
_NC = 2
_NS = 16
_NW = _NC * _NS
_K = 128
_CH = _EP // (_NW * _K)
_RPS = _NP // _NS

def _deg_body(src_hbm, dst_hbm, dout_hbm, din_hbm,
              src_v, dst_v, ones_v, zb_v, dout_sh, din_sh, sem_a, sem_b):
    c = lax.axis_index("c")
    s = lax.axis_index("s")
    wid = s * _NC + c
    base = s * _RPS

    @pl.loop(0, _RPS // 16)
    def _(i):
        zb_v[pl.ds(i * 16, 16)] = jnp.zeros((16,), jnp.float32)

    @pl.loop(0, _K // 16)
    def _(i):
        ones_v[pl.ds(i * 16, 16)] = jnp.ones((16,), jnp.float32)

    pltpu.sync_copy(zb_v, dout_sh.at[pl.ds(base, _RPS)])
    pltpu.sync_copy(zb_v, din_sh.at[pl.ds(base, _RPS)])
    plsc.subcore_barrier()

    pltpu.sync_copy(src_hbm.at[wid], src_v)
    pltpu.sync_copy(dst_hbm.at[wid], dst_v)

    @pl.loop(0, _CH)
    def _(j):
        pltpu.async_copy(ones_v, dout_sh.at[src_v.at[j]], sem_a, add=True)
        pltpu.async_copy(ones_v, din_sh.at[dst_v.at[j]], sem_b, add=True)

    @pl.loop(0, _CH)
    def _(j):
        pltpu.make_async_copy(ones_v, dout_sh.at[pl.ds(0, _K)], sem_a).wait()
        pltpu.make_async_copy(ones_v, din_sh.at[pl.ds(0, _K)], sem_b).wait()

    plsc.subcore_barrier()
    pltpu.sync_copy(dout_sh.at[pl.ds(base, _RPS)],
                    dout_hbm.at[c, pl.ds(base, _RPS)])
    pltpu.sync_copy(din_sh.at[pl.ds(base, _RPS)],
                    din_hbm.at[c, pl.ds(base, _RPS)])


def _scatter_body(h_hbm, src_hbm, dst_hbm, out_hbm,
                  src_v, dst_v, rows_a, rows_b, agg_sh, sem_a, sem_b):
    c = lax.axis_index("c")
    s = lax.axis_index("s")
    wid = s * _NC + c
    base = s * _RPS
    hch = _CH // 2

    def _issue(j, buf, sem):
        pltpu.async_copy(h_hbm.at[src_v.at[j]], buf, sem)

    def _drain(buf, sem):
        pltpu.make_async_copy(h_hbm.at[pl.ds(0, _K)], buf, sem).wait()

    pltpu.sync_copy(src_hbm.at[wid, pl.ds(0, hch)], src_v)
    _issue(0, rows_a, sem_a)
    pltpu.sync_copy(dst_hbm.at[wid, pl.ds(0, hch)], dst_v)

    @pl.loop(0, _K)
    def _(r):
        for q in range(_D // 16):
            rows_b[r, pl.ds(q * 16, 16)] = jnp.zeros((16,), jnp.float32)

    @pl.loop(0, _RPS // _K)
    def _(t):
        pltpu.sync_copy(rows_b, agg_sh.at[pl.ds(base + t * _K, _K)])
    plsc.subcore_barrier()

    @pl.loop(0, 2)
    def _(p):
        @pl.when(p > 0)
        def _():
            pltpu.sync_copy(src_hbm.at[wid, pl.ds(p * hch, hch)], src_v)
            pltpu.sync_copy(dst_hbm.at[wid, pl.ds(p * hch, hch)], dst_v)
            _issue(0, rows_a, sem_a)

        @pl.loop(0, hch, step=2)
        def _(j):
            _drain(rows_a, sem_a)
            _issue(j + 1, rows_b, sem_b)
            pltpu.sync_copy(rows_a, agg_sh.at[dst_v.at[j]], add=True)

            @pl.when(j + 2 < hch)
            def _():
                _issue(j + 2, rows_a, sem_a)

            _drain(rows_b, sem_b)
            pltpu.sync_copy(rows_b, agg_sh.at[dst_v.at[j + 1]], add=True)

    plsc.subcore_barrier()
    pltpu.sync_copy(agg_sh.at[pl.ds(base, _RPS)],
                    out_hbm.at[c, pl.ds(base, _RPS)])


@functools.cache
def _sc_kernels():
    mesh = plsc.VectorSubcoreMesh(core_axis_name="c", subcore_axis_name="s",
                                  num_cores=_NC, num_subcores=_NS)
    deg = pl.kernel(
        _deg_body,
        out_type=(jax.ShapeDtypeStruct((_NC, _NP), jnp.float32),
                  jax.ShapeDtypeStruct((_NC, _NP), jnp.float32)),
        mesh=mesh,
        scratch_types=[
            pltpu.VMEM((_CH, _K), jnp.int32),
            pltpu.VMEM((_CH, _K), jnp.int32),
            pltpu.VMEM((_K,), jnp.float32),
            pltpu.VMEM((_RPS,), jnp.float32),
            pltpu.VMEM_SHARED((_NP,), jnp.float32),
            pltpu.VMEM_SHARED((_NP,), jnp.float32),
            pltpu.SemaphoreType.DMA,
            pltpu.SemaphoreType.DMA,
        ],
    )
    scatter = pl.kernel(
        _scatter_body,
        out_type=jax.ShapeDtypeStruct((_NC, _NP, _D), jnp.float32),
        mesh=mesh,
        scratch_types=[
            pltpu.VMEM((_CH // 2, _K), jnp.int32),
            pltpu.VMEM((_CH // 2, _K), jnp.int32),
            pltpu.VMEM((_K, _D), jnp.float32),
            pltpu.VMEM((_K, _D), jnp.float32),
            pltpu.VMEM_SHARED((_NP, _D), jnp.float32),
            pltpu.SemaphoreType.DMA,
            pltpu.SemaphoreType.DMA,
        ],
    )
    return deg, scatter


_R = 1024
_G = _NP // _R


def _norm(deg):
    return jnp.where(deg > 0, lax.rsqrt(jnp.maximum(deg, 1e-12)), 0.0)


def _mm_body(x_ref, w_ref, y_ref):
    y_ref[...] = jnp.dot(x_ref[...], w_ref[...],
                         preferred_element_type=jnp.float32)


_mm = pl.pallas_call(
    _mm_body,
    grid=(_G,),
    in_specs=[pl.BlockSpec((_R, _D), lambda i: (i, 0)),
              pl.BlockSpec((_D, _D), lambda i: (0, 0))],
    out_specs=pl.BlockSpec((_R, _D), lambda i: (i, 0)),
    out_shape=jax.ShapeDtypeStruct((_NP, _D), jnp.float32),
)


def _prep_body(y_ref, do0_ref, do1_ref, di0_ref, di1_ref,
               g1_ref, ns_ref, nd_ref):
    ns = _norm(do0_ref[...] + do1_ref[...])
    nd = _norm(di0_ref[...] + di1_ref[...])
    g1_ref[...] = y_ref[...] * ns
    ns_ref[...] = ns
    nd_ref[...] = nd


_prep = pl.pallas_call(
    _prep_body,
    grid=(_G,),
    in_specs=[pl.BlockSpec((_R, _D), lambda i: (i, 0))] +
             [pl.BlockSpec((_R, 1), lambda i: (i, 0))] * 4,
    out_specs=[pl.BlockSpec((_R, _D), lambda i: (i, 0)),
               pl.BlockSpec((_R, 1), lambda i: (i, 0)),
               pl.BlockSpec((_R, 1), lambda i: (i, 0))],
    out_shape=[jax.ShapeDtypeStruct((_NP, _D), jnp.float32),
               jax.ShapeDtypeStruct((_NP, 1), jnp.float32),
               jax.ShapeDtypeStruct((_NP, 1), jnp.float32)],
)


def _mid_body(p0_ref, p1_ref, nd_ref, ns_ref, w_ref, b_ref, out_ref):
    agg = (p0_ref[...] + p1_ref[...]) * nd_ref[...]
    h = jnp.maximum(agg + b_ref[...], 0.0)
    out_ref[...] = jnp.dot(h, w_ref[...],
                           preferred_element_type=jnp.float32) * ns_ref[...]


_mid = pl.pallas_call(
    _mid_body,
    grid=(_G,),
    in_specs=[pl.BlockSpec((_R, _D), lambda i: (i, 0)),
              pl.BlockSpec((_R, _D), lambda i: (i, 0)),
              pl.BlockSpec((_R, 1), lambda i: (i, 0)),
              pl.BlockSpec((_R, 1), lambda i: (i, 0)),
              pl.BlockSpec((_D, _D), lambda i: (0, 0)),
              pl.BlockSpec((1, _D), lambda i: (0, 0))],
    out_specs=pl.BlockSpec((_R, _D), lambda i: (i, 0)),
    out_shape=jax.ShapeDtypeStruct((_NP, _D), jnp.float32),
)


def _final_body(p0_ref, p1_ref, nd_ref, b_ref, out_ref):
    i = pl.program_id(0)
    agg = (p0_ref[...] + p1_ref[...]) * nd_ref[...]
    h = jnp.maximum(agg + b_ref[...], 0.0)
    row = lax.broadcasted_iota(jnp.int32, (_R, 1), 0) + i * _R
    h = jnp.where(row < _N, h, 0.0)
    part = jnp.sum(h, axis=0, keepdims=True) * (1.0 / _N)

    @pl.when(i == 0)
    def _():
        out_ref[...] = part

    @pl.when(i > 0)
    def _():
        out_ref[...] += part


_final = pl.pallas_call(
    _final_body,
    grid=(_G,),
    in_specs=[pl.BlockSpec((_R, _D), lambda i: (i, 0)),
              pl.BlockSpec((_R, _D), lambda i: (i, 0)),
              pl.BlockSpec((_R, 1), lambda i: (i, 0)),
              pl.BlockSpec((1, _D), lambda i: (0, 0))],
    out_specs=pl.BlockSpec((1, _D), lambda i: (0, 0)),
    out_shape=jax.ShapeDtypeStruct((1, _D), jnp.float32),
)


def kernel(features, edge_index, W1, b1, W2, b2):
    src = edge_index[0].astype(jnp.int32)
    dst = edge_index[1].astype(jnp.int32)
    pad = _N + (jnp.arange(_EP - _E, dtype=jnp.int32) % (_NP - _N))
    srcp = jnp.concatenate([src, pad]).reshape(_NW, _CH, _K)
    dstp = jnp.concatenate([dst, pad]).reshape(_NW, _CH, _K)
    xp = jnp.pad(features, ((0, _NP - _N), (0, 0)))

    _deg_kernel, _scatter_kernel = _sc_kernels()
    y1 = _mm(xp, W1)
    dout_p, din_p = _deg_kernel(srcp, dstp)
    g1, ns, nd = _prep(y1,
                       dout_p[0].reshape(_NP, 1), dout_p[1].reshape(_NP, 1),
                       din_p[0].reshape(_NP, 1), din_p[1].reshape(_NP, 1))

    agg1 = _scatter_kernel(g1, srcp, dstp)
    g2 = _mid(agg1[0], agg1[1], nd, ns, W2, b1.reshape(1, _D))

    agg2 = _scatter_kernel(g2, srcp, dstp)
    return _final(agg2[0], agg2[1], nd, b2.reshape(1, _D))


# speedup vs baseline: 12.2705x; 1.0174x over previous
import functools

import jax
import jax.numpy as jnp
from jax import lax
from jax.experimental import pallas as pl
from jax.experimental.pallas import tpu as pltpu
from jax.experimental.pallas import tpu_sc as plsc

_N = 10000
_D = 128
_E = 320000
_NP = 10240
_EP = 327680
_NC = 2
_NS = 16
_NW = _NC * _NS
_K = 128
_CH = _EP // (_NW * _K)
_RPS = _NP // _NS

def _deg_body(src_hbm, dst_hbm, one_src_hbm, one_dst_hbm, z8_hbm, deg_hbm,
              src_v, dst_v, one_src_v, one_dst_v, zb_v, deg_sh,
              sem_a, sem_b):
    c = lax.axis_index("c")
    s = lax.axis_index("s")
    wid = s * _NC + c
    base = s * _RPS

    pltpu.sync_copy(one_src_hbm, one_src_v)
    pltpu.sync_copy(one_dst_hbm, one_dst_v)
    pltpu.sync_copy(z8_hbm, zb_v)
    pltpu.sync_copy(zb_v, deg_sh.at[pl.ds(base, _RPS)])
    plsc.subcore_barrier()

    pltpu.sync_copy(src_hbm.at[wid], src_v)
    pltpu.sync_copy(dst_hbm.at[wid], dst_v)

    @pl.loop(0, _CH)
    def _(j):
        pltpu.async_copy(one_src_v, deg_sh.at[src_v.at[j]], sem_a, add=True)
        pltpu.async_copy(one_dst_v, deg_sh.at[dst_v.at[j]], sem_b, add=True)

    @pl.loop(0, _CH)
    def _(j):
        pltpu.make_async_copy(one_src_v, deg_sh.at[pl.ds(0, _K)], sem_a).wait()
        pltpu.make_async_copy(one_dst_v, deg_sh.at[pl.ds(0, _K)], sem_b).wait()

    plsc.subcore_barrier()
    pltpu.sync_copy(deg_sh.at[pl.ds(base, _RPS)],
                    deg_hbm.at[c, pl.ds(base, _RPS)])


def _scatter_body(h_hbm, src_hbm, dst_hbm, z_hbm, out_hbm,
                  src_v, dst_v, rows_a, rows_b, agg_sh, sem_a, sem_b):
    c = lax.axis_index("c")
    s = lax.axis_index("s")
    wid = s * _NC + c
    base = s * _RPS
    hch = _CH // 2

    def _issue(j, buf, sem):
        pltpu.async_copy(h_hbm.at[src_v.at[j]], buf, sem)

    def _drain(buf, sem):
        pltpu.make_async_copy(h_hbm.at[pl.ds(0, _K)], buf, sem).wait()

    pltpu.sync_copy(src_hbm.at[wid, pl.ds(0, hch)], src_v)
    _issue(0, rows_a, sem_a)
    pltpu.sync_copy(dst_hbm.at[wid, pl.ds(0, hch)], dst_v)

    pltpu.sync_copy(z_hbm, rows_b)

    @pl.loop(0, _RPS // _K)
    def _(t):
        pltpu.sync_copy(rows_b, agg_sh.at[pl.ds(base + t * _K, _K)])
    plsc.subcore_barrier()

    @pl.loop(0, 2)
    def _(p):
        @pl.when(p > 0)
        def _():
            pltpu.sync_copy(src_hbm.at[wid, pl.ds(p * hch, hch)], src_v)
            pltpu.sync_copy(dst_hbm.at[wid, pl.ds(p * hch, hch)], dst_v)
            _issue(0, rows_a, sem_a)

        @pl.loop(0, hch, step=2)
        def _(j):
            _drain(rows_a, sem_a)
            _issue(j + 1, rows_b, sem_b)
            pltpu.sync_copy(rows_a, agg_sh.at[dst_v.at[j]], add=True)

            @pl.when(j + 2 < hch)
            def _():
                _issue(j + 2, rows_a, sem_a)

            _drain(rows_b, sem_b)
            pltpu.sync_copy(rows_b, agg_sh.at[dst_v.at[j + 1]], add=True)

    plsc.subcore_barrier()
    pltpu.sync_copy(agg_sh.at[pl.ds(base, _RPS)],
                    out_hbm.at[c, pl.ds(base, _RPS)])


@functools.cache
def _sc_kernels():
    mesh = plsc.VectorSubcoreMesh(core_axis_name="c", subcore_axis_name="s",
                                  num_cores=_NC, num_subcores=_NS)
    deg = pl.kernel(
        _deg_body,
        out_type=jax.ShapeDtypeStruct((_NC, _NP, 8), jnp.float32),
        mesh=mesh,
        scratch_types=[
            pltpu.VMEM((_CH, _K), jnp.int32),
            pltpu.VMEM((_CH, _K), jnp.int32),
            pltpu.VMEM((_K, 8), jnp.float32),
            pltpu.VMEM((_K, 8), jnp.float32),
            pltpu.VMEM((_RPS, 8), jnp.float32),
            pltpu.VMEM_SHARED((_NP, 8), jnp.float32),
            pltpu.SemaphoreType.DMA,
            pltpu.SemaphoreType.DMA,
        ],
        compiler_params=pltpu.CompilerParams(use_tc_tiling_on_sc=False),
    )
    scatter = pl.kernel(
        _scatter_body,
        out_type=jax.ShapeDtypeStruct((_NC, _NP, _D), jnp.float32),
        mesh=mesh,
        scratch_types=[
            pltpu.VMEM((_CH // 2, _K), jnp.int32),
            pltpu.VMEM((_CH // 2, _K), jnp.int32),
            pltpu.VMEM((_K, _D), jnp.float32),
            pltpu.VMEM((_K, _D), jnp.float32),
            pltpu.VMEM_SHARED((_NP, _D), jnp.float32),
            pltpu.SemaphoreType.DMA,
            pltpu.SemaphoreType.DMA,
        ],
    )
    return deg, scatter


_R = 1024
_G = _NP // _R


def _norm(deg):
    return jnp.where(deg > 0, lax.rsqrt(jnp.maximum(deg, 1e-12)), 0.0)


_DSPEC0 = pl.BlockSpec((1, _R, 8), lambda i: (0, i, 0))
_DSPEC1 = pl.BlockSpec((1, _R, 8), lambda i: (1, i, 0))
_PSPEC0 = pl.BlockSpec((1, _R, _D), lambda i: (0, i, 0))
_PSPEC1 = pl.BlockSpec((1, _R, _D), lambda i: (1, i, 0))


def _prep_body(x_ref, w_ref, d0_ref, d1_ref, g1_ref):
    deg = d0_ref[...][0] + d1_ref[...][0]
    ns = _norm(deg[:, 0:1])
    y = jnp.dot(x_ref[...], w_ref[...], preferred_element_type=jnp.float32)
    g1_ref[...] = y * ns


_prep = pl.pallas_call(
    _prep_body,
    grid=(_G,),
    in_specs=[pl.BlockSpec((_R, _D), lambda i: (i, 0)),
              pl.BlockSpec((_D, _D), lambda i: (0, 0)),
              _DSPEC0, _DSPEC1],
    out_specs=pl.BlockSpec((_R, _D), lambda i: (i, 0)),
    out_shape=jax.ShapeDtypeStruct((_NP, _D), jnp.float32),
)


def _mid_body(p0_ref, p1_ref, d0_ref, d1_ref, w_ref, b_ref, out_ref):
    deg = d0_ref[...][0] + d1_ref[...][0]
    ns = _norm(deg[:, 0:1])
    nd = _norm(deg[:, 4:5])
    agg = (p0_ref[...][0] + p1_ref[...][0]) * nd
    h = jnp.maximum(agg + b_ref[...], 0.0)
    out_ref[...] = jnp.dot(h, w_ref[...],
                           preferred_element_type=jnp.float32) * ns


_mid = pl.pallas_call(
    _mid_body,
    grid=(_G,),
    in_specs=[_PSPEC0, _PSPEC1, _DSPEC0, _DSPEC1,
              pl.BlockSpec((_D, _D), lambda i: (0, 0)),
              pl.BlockSpec((1, _D), lambda i: (0, 0))],
    out_specs=pl.BlockSpec((_R, _D), lambda i: (i, 0)),
    out_shape=jax.ShapeDtypeStruct((_NP, _D), jnp.float32),
)


def _final_body(p0_ref, p1_ref, d0_ref, d1_ref, b_ref, out_ref):
    i = pl.program_id(0)
    deg = d0_ref[...][0] + d1_ref[...][0]
    nd = _norm(deg[:, 4:5])
    agg = (p0_ref[...][0] + p1_ref[...][0]) * nd
    h = jnp.maximum(agg + b_ref[...], 0.0)
    row = lax.broadcasted_iota(jnp.int32, (_R, 1), 0) + i * _R
    h = jnp.where(row < _N, h, 0.0)
    part = jnp.sum(h, axis=0, keepdims=True) * (1.0 / _N)

    @pl.when(i == 0)
    def _():
        out_ref[...] = part

    @pl.when(i > 0)
    def _():
        out_ref[...] += part


_final = pl.pallas_call(
    _final_body,
    grid=(_G,),
    in_specs=[_PSPEC0, _PSPEC1, _DSPEC0, _DSPEC1,
              pl.BlockSpec((1, _D), lambda i: (0, 0))],
    out_specs=pl.BlockSpec((1, _D), lambda i: (0, 0)),
    out_shape=jax.ShapeDtypeStruct((1, _D), jnp.float32),
)


def kernel(features, edge_index, W1, b1, W2, b2):
    src = edge_index[0].astype(jnp.int32)
    dst = edge_index[1].astype(jnp.int32)
    pad = _N + (jnp.arange(_EP - _E, dtype=jnp.int32) % (_NP - _N))
    srcp = jnp.concatenate([src, pad]).reshape(_NW, _CH, _K)
    dstp = jnp.concatenate([dst, pad]).reshape(_NW, _CH, _K)
    xp = jnp.pad(features, ((0, _NP - _N), (0, 0)))
    one_src = jnp.zeros((_K, 8), jnp.float32).at[:, 0].set(1.0)
    one_dst = jnp.zeros((_K, 8), jnp.float32).at[:, 4].set(1.0)
    z8 = jnp.zeros((_RPS, 8), jnp.float32)
    zkd = jnp.zeros((_K, _D), jnp.float32)

    _deg_kernel, _scatter_kernel = _sc_kernels()
    deg_p = _deg_kernel(srcp, dstp, one_src, one_dst, z8)
    g1 = _prep(xp, W1, deg_p, deg_p)

    agg1 = _scatter_kernel(g1, srcp, dstp, zkd)
    g2 = _mid(agg1, agg1, deg_p, deg_p, W2, b1.reshape(1, _D))

    agg2 = _scatter_kernel(g2, srcp, dstp, zkd)
    return _final(agg2, agg2, deg_p, deg_p, b2.reshape(1, _D))


# speedup vs baseline: 12.5759x; 1.0249x over previous
import functools

import numpy as np

import jax
import jax.numpy as jnp
from jax import lax
from jax.experimental import pallas as pl
from jax.experimental.pallas import tpu as pltpu
from jax.experimental.pallas import tpu_sc as plsc

_N = 10000
_D = 128
_E = 320000
_NP = 10240
_EP = 327680
_NC = 2
_NS = 16
_NW = _NC * _NS
_K = 128
_CH = _EP // (_NW * _K)
_RPS = _NP // _NS

_PAD_IDX = np.asarray(_N + np.arange(_EP - _E) % (_NP - _N), np.int32)

def _deg_body(src_hbm, dst_hbm, one_src_hbm, one_dst_hbm, z8_hbm, deg_hbm,
              src_v, dst_v, one_src_v, one_dst_v, zb_v, deg_sh,
              sem_a, sem_b):
    c = lax.axis_index("c")
    s = lax.axis_index("s")
    wid = s * _NC + c
    base = s * _RPS

    pltpu.sync_copy(one_src_hbm, one_src_v)
    pltpu.sync_copy(one_dst_hbm, one_dst_v)
    pltpu.sync_copy(z8_hbm, zb_v)
    pltpu.sync_copy(zb_v, deg_sh.at[pl.ds(base, _RPS)])
    plsc.subcore_barrier()

    pltpu.sync_copy(src_hbm.at[wid], src_v)
    pltpu.sync_copy(dst_hbm.at[wid], dst_v)

    @pl.loop(0, _CH)
    def _(j):
        pltpu.async_copy(one_src_v, deg_sh.at[src_v.at[j]], sem_a, add=True)
        pltpu.async_copy(one_dst_v, deg_sh.at[dst_v.at[j]], sem_b, add=True)

    @pl.loop(0, _CH)
    def _(j):
        pltpu.make_async_copy(one_src_v, deg_sh.at[pl.ds(0, _K)], sem_a).wait()
        pltpu.make_async_copy(one_dst_v, deg_sh.at[pl.ds(0, _K)], sem_b).wait()

    plsc.subcore_barrier()
    pltpu.sync_copy(deg_sh.at[pl.ds(base, _RPS)],
                    deg_hbm.at[c, pl.ds(base, _RPS)])


def _scatter_body(h_hbm, src_hbm, dst_hbm, z_hbm, out_hbm,
                  src_v, dst_v, rows_a, rows_b, agg_sh, sem_a, sem_b):
    c = lax.axis_index("c")
    s = lax.axis_index("s")
    wid = s * _NC + c
    base = s * _RPS
    hch = _CH // 2

    def _issue(j, buf, sem):
        pltpu.async_copy(h_hbm.at[src_v.at[j]], buf, sem)

    def _drain(buf, sem):
        pltpu.make_async_copy(h_hbm.at[pl.ds(0, _K)], buf, sem).wait()

    pltpu.sync_copy(src_hbm.at[wid, pl.ds(0, hch)], src_v)
    _issue(0, rows_a, sem_a)
    pltpu.sync_copy(dst_hbm.at[wid, pl.ds(0, hch)], dst_v)

    pltpu.sync_copy(z_hbm, rows_b)

    @pl.loop(0, _RPS // _K)
    def _(t):
        pltpu.sync_copy(rows_b, agg_sh.at[pl.ds(base + t * _K, _K)])
    plsc.subcore_barrier()

    @pl.loop(0, 2)
    def _(p):
        @pl.when(p > 0)
        def _():
            pltpu.sync_copy(src_hbm.at[wid, pl.ds(p * hch, hch)], src_v)
            pltpu.sync_copy(dst_hbm.at[wid, pl.ds(p * hch, hch)], dst_v)
            _issue(0, rows_a, sem_a)

        @pl.loop(0, hch, step=2)
        def _(j):
            _drain(rows_a, sem_a)
            _issue(j + 1, rows_b, sem_b)
            pltpu.sync_copy(rows_a, agg_sh.at[dst_v.at[j]], add=True)

            @pl.when(j + 2 < hch)
            def _():
                _issue(j + 2, rows_a, sem_a)

            _drain(rows_b, sem_b)
            pltpu.sync_copy(rows_b, agg_sh.at[dst_v.at[j + 1]], add=True)

    plsc.subcore_barrier()
    pltpu.sync_copy(agg_sh.at[pl.ds(base, _RPS)],
                    out_hbm.at[c, pl.ds(base, _RPS)])


@functools.cache
def _sc_kernels():
    mesh = plsc.VectorSubcoreMesh(core_axis_name="c", subcore_axis_name="s",
                                  num_cores=_NC, num_subcores=_NS)
    deg = pl.kernel(
        _deg_body,
        out_type=jax.ShapeDtypeStruct((_NC, _NP, 8), jnp.float32),
        mesh=mesh,
        scratch_types=[
            pltpu.VMEM((_CH, _K), jnp.int32),
            pltpu.VMEM((_CH, _K), jnp.int32),
            pltpu.VMEM((_K, 8), jnp.float32),
            pltpu.VMEM((_K, 8), jnp.float32),
            pltpu.VMEM((_RPS, 8), jnp.float32),
            pltpu.VMEM_SHARED((_NP, 8), jnp.float32),
            pltpu.SemaphoreType.DMA,
            pltpu.SemaphoreType.DMA,
        ],
        compiler_params=pltpu.CompilerParams(use_tc_tiling_on_sc=False),
    )
    scatter = pl.kernel(
        _scatter_body,
        out_type=jax.ShapeDtypeStruct((_NC, _NP, _D), jnp.float32),
        mesh=mesh,
        scratch_types=[
            pltpu.VMEM((_CH // 2, _K), jnp.int32),
            pltpu.VMEM((_CH // 2, _K), jnp.int32),
            pltpu.VMEM((_K, _D), jnp.float32),
            pltpu.VMEM((_K, _D), jnp.float32),
            pltpu.VMEM_SHARED((_NP, _D), jnp.float32),
            pltpu.SemaphoreType.DMA,
            pltpu.SemaphoreType.DMA,
        ],
    )
    return deg, scatter


_R = 2048
_G = _NP // _R


def _norm(deg):
    return jnp.where(deg > 0, lax.rsqrt(jnp.maximum(deg, 1e-12)), 0.0)


_DSPEC0 = pl.BlockSpec((1, _R, 8), lambda i: (0, i, 0))
_DSPEC1 = pl.BlockSpec((1, _R, 8), lambda i: (1, i, 0))
_PSPEC0 = pl.BlockSpec((1, _R, _D), lambda i: (0, i, 0))
_PSPEC1 = pl.BlockSpec((1, _R, _D), lambda i: (1, i, 0))


def _prep_body(x_ref, w_ref, d0_ref, d1_ref, g1_ref):
    deg = d0_ref[...][0] + d1_ref[...][0]
    ns = _norm(deg[:, 0:1])
    y = jnp.dot(x_ref[...], w_ref[...], preferred_element_type=jnp.float32)
    g1_ref[...] = y * ns


_prep = pl.pallas_call(
    _prep_body,
    grid=(_G,),
    in_specs=[pl.BlockSpec((_R, _D), lambda i: (i, 0)),
              pl.BlockSpec((_D, _D), lambda i: (0, 0)),
              _DSPEC0, _DSPEC1],
    out_specs=pl.BlockSpec((_R, _D), lambda i: (i, 0)),
    out_shape=jax.ShapeDtypeStruct((_NP, _D), jnp.float32),
)


def _mid_body(p0_ref, p1_ref, d0_ref, d1_ref, w_ref, b_ref, out_ref):
    deg = d0_ref[...][0] + d1_ref[...][0]
    ns = _norm(deg[:, 0:1])
    nd = _norm(deg[:, 4:5])
    agg = (p0_ref[...][0] + p1_ref[...][0]) * nd
    h = jnp.maximum(agg + b_ref[...], 0.0)
    out_ref[...] = jnp.dot(h, w_ref[...],
                           preferred_element_type=jnp.float32) * ns


_mid = pl.pallas_call(
    _mid_body,
    grid=(_G,),
    in_specs=[_PSPEC0, _PSPEC1, _DSPEC0, _DSPEC1,
              pl.BlockSpec((_D, _D), lambda i: (0, 0)),
              pl.BlockSpec((1, _D), lambda i: (0, 0))],
    out_specs=pl.BlockSpec((_R, _D), lambda i: (i, 0)),
    out_shape=jax.ShapeDtypeStruct((_NP, _D), jnp.float32),
)


def _final_body(p0_ref, p1_ref, d0_ref, d1_ref, b_ref, out_ref):
    i = pl.program_id(0)
    deg = d0_ref[...][0] + d1_ref[...][0]
    nd = _norm(deg[:, 4:5])
    agg = (p0_ref[...][0] + p1_ref[...][0]) * nd
    h = jnp.maximum(agg + b_ref[...], 0.0)
    row = lax.broadcasted_iota(jnp.int32, (_R, 1), 0) + i * _R
    h = jnp.where(row < _N, h, 0.0)
    part = jnp.sum(h, axis=0, keepdims=True) * (1.0 / _N)

    @pl.when(i == 0)
    def _():
        out_ref[...] = part

    @pl.when(i > 0)
    def _():
        out_ref[...] += part


_final = pl.pallas_call(
    _final_body,
    grid=(_G,),
    in_specs=[_PSPEC0, _PSPEC1, _DSPEC0, _DSPEC1,
              pl.BlockSpec((1, _D), lambda i: (0, 0))],
    out_specs=pl.BlockSpec((1, _D), lambda i: (0, 0)),
    out_shape=jax.ShapeDtypeStruct((1, _D), jnp.float32),
)


def kernel(features, edge_index, W1, b1, W2, b2):
    src = edge_index[0].astype(jnp.int32)
    dst = edge_index[1].astype(jnp.int32)
    pad = jnp.asarray(_PAD_IDX)
    srcp = jnp.concatenate([src, pad]).reshape(_NW, _CH, _K)
    dstp = jnp.concatenate([dst, pad]).reshape(_NW, _CH, _K)
    xp = jnp.pad(features, ((0, _NP - _N), (0, 0)))
    one_src = jnp.zeros((_K, 8), jnp.float32).at[:, 0].set(1.0)
    one_dst = jnp.zeros((_K, 8), jnp.float32).at[:, 4].set(1.0)
    z8 = jnp.zeros((_RPS, 8), jnp.float32)
    zkd = jnp.zeros((_K, _D), jnp.float32)

    _deg_kernel, _scatter_kernel = _sc_kernels()
    deg_p = _deg_kernel(srcp, dstp, one_src, one_dst, z8)
    g1 = _prep(xp, W1, deg_p, deg_p)

    agg1 = _scatter_kernel(g1, srcp, dstp, zkd)
    g2 = _mid(agg1, agg1, deg_p, deg_p, W2, b1.reshape(1, _D))

    agg2 = _scatter_kernel(g2, srcp, dstp, zkd)
    return _final(agg2, agg2, deg_p, deg_p, b2.reshape(1, _D))
